# baseline (speedup 1.0000x reference)
import functools

import jax
import jax.numpy as jnp
from jax import lax
from jax.experimental import pallas as pl
from jax.experimental.pallas import tpu as pltpu
from jax.experimental.pallas import tpu_sc as plsc

_NC = 2
_NS = 16
_NW = _NC * _NS

_BATCH = 16384
_HEADS = 26
_DIM = 32
_N = _BATCH * _HEADS
_PER_W = _N // _NW
_ROWS_W = _PER_W // 128
_CH = 128
_NCHUNK = _PER_W // _CH
_OUT_RW = _CH * _DIM // 128
_LANES = 16


def _body(hash_hbm, table_hbm, off_hbm, out_hbm,
          idx_v, col_v, off_v, g_buf, o_buf, gsem0, gsem1, wsem0, wsem1):
    wid = lax.axis_index("s") * _NC + lax.axis_index("c")

    pltpu.sync_copy(hash_hbm.at[pl.ds(wid * _ROWS_W, _ROWS_W)], idx_v)
    pltpu.sync_copy(off_hbm, off_v)

    def _prep(r, carry):
        q = lax.rem(r, 13)
        for p in range(128 // _LANES):
            c0 = p * _LANES
            v = idx_v[r, pl.ds(c0, _LANES)] + off_v[q, pl.ds(c0, _LANES)]
            col_v[pl.ds(r * 128 + c0, _LANES)] = (v & 3) * _DIM
            idx_v[r, pl.ds(c0, _LANES)] = lax.shift_right_logical(v, 2)
        return carry

    lax.fori_loop(0, _ROWS_W, _prep, 0)

    gsems = (gsem0, gsem1)
    wsems = (wsem0, wsem1)
    out_base = wid * (_NCHUNK * _OUT_RW)

    def g_copy(c, s):
        return pltpu.make_async_copy(
            table_hbm.at[idx_v.at[c]], g_buf.at[s], gsems[s])

    def w_copy(c, s):
        return pltpu.make_async_copy(
            o_buf.at[s], out_hbm.at[pl.ds(out_base + c * _OUT_RW, _OUT_RW)],
            wsems[s])

    def rearrange(c, s):
        def _rb(rb, carry):
            colv = col_v[pl.ds(c * _CH + rb * _LANES, _LANES)]
            for t in range(_LANES):
                col = colv[t]
                src = g_buf.at[s].at[rb * _LANES + t]
                v0 = src[pl.ds(col, _LANES)]
                v1 = src[pl.ds(col + _LANES, _LANES)]
                orow = o_buf.at[s].at[rb * 4 + t // 4]
                orow[pl.ds((t % 4) * _DIM, _LANES)] = v0
                orow[pl.ds((t % 4) * _DIM + _LANES, _LANES)] = v1
            return carry

        lax.fori_loop(0, _CH // _LANES, _rb, 0)

    g_copy(0, 0).start()

    def _pair(c2, carry):
        c0 = c2 * 2
        c1 = c0 + 1
        g_copy(c0, 0).wait()
        g_copy(c1, 1).start()

        @pl.when(c2 > 0)
        def _():
            w_copy(c0 - 2, 0).wait()

        rearrange(c0, 0)
        w_copy(c0, 0).start()

        g_copy(c1, 1).wait()

        @pl.when(c2 < _NCHUNK // 2 - 1)
        def _():
            g_copy(c0 + 2, 0).start()

        @pl.when(c2 > 0)
        def _():
            w_copy(c1 - 2, 1).wait()

        rearrange(c1, 1)
        w_copy(c1, 1).start()
        return carry

    lax.fori_loop(0, _NCHUNK // 2, _pair, 0)
    w_copy(_NCHUNK - 2, 0).wait()
    w_copy(_NCHUNK - 1, 1).wait()


def _gather(hash128, table128, off128):
    mesh = plsc.VectorSubcoreMesh(core_axis_name="c", subcore_axis_name="s")
    k = functools.partial(
        pl.kernel,
        mesh=mesh,
        out_type=jax.ShapeDtypeStruct((_N * _DIM // 128, 128), jnp.float32),
        scratch_types=[
            pltpu.VMEM((_ROWS_W, 128), jnp.int32),
            pltpu.VMEM((_PER_W,), jnp.int32),
            pltpu.VMEM((13, 128), jnp.int32),
            pltpu.VMEM((2, _CH, 128), jnp.float32),
            pltpu.VMEM((2, _OUT_RW, 128), jnp.float32),
            pltpu.SemaphoreType.DMA,
            pltpu.SemaphoreType.DMA,
            pltpu.SemaphoreType.DMA,
            pltpu.SemaphoreType.DMA,
        ],
    )(_body)
    return k(hash128, table128, off128)


def kernel(hash_ids, table, offsets):
    hash128 = hash_ids.reshape(_N // 128, 128)
    table128 = table.reshape(table.shape[0] * _DIM // 128, 128)
    off128 = jnp.tile(offsets, 64).reshape(13, 128)
    out = _gather(hash128, table128, off128)
    return out.reshape(_BATCH, _HEADS, _DIM)
